# baseline (speedup 1.0000x reference)
import functools

import jax
import jax.numpy as jnp
from jax.experimental import pallas as pl
from jax.experimental.pallas import tpu as pltpu
from jax.experimental.pallas import tpu_sc as plsc

_RHO = 50.0
_B1 = 0.1
_TOPK = 50
_N = 471
_D = 65536
_NPAD = 480
_NV = _NPAD // 16
_DBLK = 4096


_COLS = ((0, 256), (256, _N - 256))


def _tc_body(theta_ref, q_ref, u_ref, a_ref, out_ref, bits_ref, acc_ref,
             s_ref, p0_ref, p1_ref):
    i = pl.program_id(0)
    hp = jax.lax.Precision.HIGHEST

    def _mm(m, r):
        return jnp.dot(m, r, preferred_element_type=jnp.float32,
                       precision=hp)

    @pl.when(i == 0)
    def _init():
        acc_ref[...] = jnp.zeros_like(acc_ref)

    acc_ref[...] += jnp.dot(theta_ref[...], q_ref[...],
                            preferred_element_type=jnp.float32)

    def mk_start(c0, w):
        def f():
            b = -a_ref[...] / _RHO
            if c0 == 0:
                rows = jax.lax.broadcasted_iota(jnp.int32, (_N, _N), 0)
                cols = jax.lax.broadcasted_iota(jnp.int32, (_N, _N), 1)
                s_ref[...] = jnp.where(rows == cols, 1.0, 0.0) + b
            p0_ref[:, pl.ds(c0, w)] = _mm(b, b[:, c0:c0 + w])
        return f

    def mk_supd(pin_ref, c0, w):
        def f():
            sl = (slice(None), pl.ds(c0, w))
            s_ref[sl] += _mm(pin_ref[...], s_ref[sl])
        return f

    def mk_pupd(pin_ref, pout_ref, c0, w):
        def f():
            pout_ref[:, pl.ds(c0, w)] = _mm(pin_ref[...],
                                            pin_ref[:, pl.ds(c0, w)])
        return f

    ops = [mk_start(c0, w) for c0, w in _COLS]
    pair = (p0_ref, p1_ref)
    for j in range(7):
        pin, pout = pair[j % 2], pair[(j + 1) % 2]
        ops += [mk_supd(pin, c0, w) for c0, w in _COLS]
        if j < 6:
            ops += [mk_pupd(pin, pout, c0, w) for c0, w in _COLS]
    for k, op in enumerate(ops):
        pl.when(i == k // 2)(op)

    @pl.when(i == pl.num_programs(0) - 1)
    def _finish():
        v = _B1 + _RHO * (acc_ref[...] * (1.0 / _N) - u_ref[...])
        temp = s_ref[...] / _RHO
        z = jnp.dot(temp, v, preferred_element_type=jnp.float32)
        zp = jnp.concatenate(
            [jnp.maximum(z, 0.0), jnp.zeros((_NPAD - _N, 1), jnp.float32)],
            axis=0)
        out_ref[...] = zp
        bits_ref[...] = jax.lax.bitcast_convert_type(zp, jnp.int32)


def _tc_stage(theta, u, A_Transfer, q_t):
    grid = (_D // _DBLK,)
    return pl.pallas_call(
        _tc_body,
        grid=grid,
        in_specs=[
            pl.BlockSpec((_N, _DBLK), lambda i: (0, i)),
            pl.BlockSpec((_DBLK, 1), lambda i: (i, 0)),
            pl.BlockSpec((_N, 1), lambda i: (0, 0)),
            pl.BlockSpec((_N, _N), lambda i: (0, 0)),
        ],
        out_specs=[
            pl.BlockSpec((_NPAD, 1), lambda i: (0, 0)),
            pl.BlockSpec((_NPAD, 1), lambda i: (0, 0)),
        ],
        out_shape=[
            jax.ShapeDtypeStruct((_NPAD, 1), jnp.float32),
            jax.ShapeDtypeStruct((_NPAD, 1), jnp.int32),
        ],
        scratch_shapes=[
            pltpu.VMEM((_N, 1), jnp.float32),
            pltpu.VMEM((_N, _N), jnp.float32),
            pltpu.VMEM((_N, _N), jnp.float32),
            pltpu.VMEM((_N, _N), jnp.float32),
        ],
        compiler_params=pltpu.CompilerParams(
            dimension_semantics=("arbitrary",)),
    )(theta, q_t.reshape(_D, 1), u.reshape(_N, 1), A_Transfer)


def _sc_topk_body(z_hbm, zb_hbm, out_hbm, z_v, zb_v, o_v):
    wid = jax.lax.axis_index("c") * 16 + jax.lax.axis_index("s")

    @pl.when(wid == 0)
    def _():
        pltpu.sync_copy(z_hbm, z_v)
        pltpu.sync_copy(zb_hbm, zb_v)

        def count_gt(bits):
            tvec = jnp.full((16,), bits, jnp.int32)

            def jstep(j, cvec):
                zj = zb_v[pl.ds(j * 16, 16)]
                return cvec + jnp.where(zj > tvec, jnp.int32(1), jnp.int32(0))

            cvec = jax.lax.fori_loop(
                0, _NV, jstep, jnp.zeros((16,), jnp.int32))
            return jnp.sum(cvec)

        def bstep(_, lohi):
            lo, hi = lohi
            mid = lo + jax.lax.shift_right_logical(hi - lo, 1)
            c = count_gt(mid)
            lt = c < _TOPK
            return (jnp.where(lt, lo, mid + 1), jnp.where(lt, mid, hi))

        lo, hi = jax.lax.fori_loop(
            0, 31, bstep, (jnp.int32(0), jnp.int32(0x7F800000)))
        tbits = lo
        n_gt = count_gt(tbits)
        need = _TOPK - n_gt

        tvec = jnp.full((16,), tbits, jnp.int32)

        def mstep(j, carry):
            svec, eqc = carry
            zj = z_v[pl.ds(j * 16, 16)]
            zbj = zb_v[pl.ds(j * 16, 16)]
            gt = zbj > tvec
            eqi = jnp.where(zbj == tvec, jnp.int32(1), jnp.int32(0))
            rank = (plsc.cumsum(eqi) - eqi) + eqc
            keep = jnp.logical_or(gt, jnp.logical_and(eqi > 0, rank < need))
            mz = jnp.where(keep, zj, 0.0)
            o_v[pl.ds(j * 16, 16)] = mz
            return (svec + mz, eqc + jnp.sum(eqi))

        svec, _ = jax.lax.fori_loop(
            0, _NV, mstep, (jnp.zeros((16,), jnp.float32), jnp.int32(0)))
        total = jnp.sum(svec)
        inv_vec = 1.0 / (jnp.full((16,), total) + 1e-8)

        def nstep(j, c):
            o_v[pl.ds(j * 16, 16)] = o_v[pl.ds(j * 16, 16)] * inv_vec
            return c

        jax.lax.fori_loop(0, _NV, nstep, 0)
        pltpu.sync_copy(o_v, out_hbm)


def _sc_stage(z_pad, z_bits):
    mesh = plsc.VectorSubcoreMesh(core_axis_name="c", subcore_axis_name="s")
    call = pl.kernel(
        _sc_topk_body,
        out_type=jax.ShapeDtypeStruct((_NPAD,), jnp.float32),
        mesh=mesh,
        scratch_types=[
            pltpu.VMEM((_NPAD,), jnp.float32),
            pltpu.VMEM((_NPAD,), jnp.int32),
            pltpu.VMEM((_NPAD,), jnp.float32),
        ],
        compiler_params=pltpu.CompilerParams(needs_layout_passes=False),
    )
    return call(z_pad, z_bits)


def kernel(theta, u, A_Transfer, q_t):
    z_pad, z_bits = _tc_stage(theta, u, A_Transfer, q_t)
    out = _sc_stage(z_pad.reshape(_NPAD), z_bits.reshape(_NPAD))
    return out[:_N]


# speedup vs baseline: 1.0592x; 1.0592x over previous
import functools

import jax
import jax.numpy as jnp
from jax.experimental import pallas as pl
from jax.experimental.pallas import tpu as pltpu
from jax.experimental.pallas import tpu_sc as plsc

_RHO = 50.0
_B1 = 0.1
_TOPK = 50
_N = 471
_D = 65536
_NPAD = 480
_NV = _NPAD // 16
_DBLK = 8192


_COLS = ((0, 256), (256, _N - 256))


def _tc_body(theta_ref, q_ref, u_ref, a_ref, out_ref, bits_ref, acc_ref,
             s_ref, p0_ref, p1_ref):
    i = pl.program_id(0)
    hp = jax.lax.Precision.HIGHEST

    def _mm(m, r):
        return jnp.dot(m, r, preferred_element_type=jnp.float32,
                       precision=hp)

    @pl.when(i == 0)
    def _init():
        acc_ref[...] = jnp.zeros_like(acc_ref)

    acc_ref[...] += jnp.dot(theta_ref[...], q_ref[...],
                            preferred_element_type=jnp.float32)

    def mk_start(c0, w):
        def f():
            b = -a_ref[...] / _RHO
            if c0 == 0:
                rows = jax.lax.broadcasted_iota(jnp.int32, (_N, _N), 0)
                cols = jax.lax.broadcasted_iota(jnp.int32, (_N, _N), 1)
                s_ref[...] = jnp.where(rows == cols, 1.0, 0.0) + b
            p0_ref[:, pl.ds(c0, w)] = _mm(b, b[:, c0:c0 + w])
        return f

    def mk_supd(pin_ref, c0, w):
        def f():
            sl = (slice(None), pl.ds(c0, w))
            s_ref[sl] += _mm(pin_ref[...], s_ref[sl])
        return f

    def mk_pupd(pin_ref, pout_ref, c0, w):
        def f():
            pout_ref[:, pl.ds(c0, w)] = _mm(pin_ref[...],
                                            pin_ref[:, pl.ds(c0, w)])
        return f

    ops = [mk_start(c0, w) for c0, w in _COLS]
    pair = (p0_ref, p1_ref)
    for j in range(7):
        pin, pout = pair[j % 2], pair[(j + 1) % 2]
        ops += [mk_supd(pin, c0, w) for c0, w in _COLS]
        if j < 6:
            ops += [mk_pupd(pin, pout, c0, w) for c0, w in _COLS]
    for k, op in enumerate(ops):
        pl.when(i == k // 4)(op)

    @pl.when(i == pl.num_programs(0) - 1)
    def _finish():
        v = _B1 + _RHO * (acc_ref[...] * (1.0 / _N) - u_ref[...])
        temp = s_ref[...] / _RHO
        z = jnp.dot(temp, v, preferred_element_type=jnp.float32)
        zp = jnp.concatenate(
            [jnp.maximum(z, 0.0), jnp.zeros((_NPAD - _N, 1), jnp.float32)],
            axis=0)
        out_ref[...] = zp
        bits_ref[...] = jax.lax.bitcast_convert_type(zp, jnp.int32)


def _tc_stage(theta, u, A_Transfer, q_t):
    grid = (_D // _DBLK,)
    return pl.pallas_call(
        _tc_body,
        grid=grid,
        in_specs=[
            pl.BlockSpec((_N, _DBLK), lambda i: (0, i)),
            pl.BlockSpec((_DBLK, 1), lambda i: (i, 0)),
            pl.BlockSpec((_N, 1), lambda i: (0, 0)),
            pl.BlockSpec((_N, _N), lambda i: (0, 0)),
        ],
        out_specs=[
            pl.BlockSpec((_NPAD, 1), lambda i: (0, 0)),
            pl.BlockSpec((_NPAD, 1), lambda i: (0, 0)),
        ],
        out_shape=[
            jax.ShapeDtypeStruct((_NPAD, 1), jnp.float32),
            jax.ShapeDtypeStruct((_NPAD, 1), jnp.int32),
        ],
        scratch_shapes=[
            pltpu.VMEM((_N, 1), jnp.float32),
            pltpu.VMEM((_N, _N), jnp.float32),
            pltpu.VMEM((_N, _N), jnp.float32),
            pltpu.VMEM((_N, _N), jnp.float32),
        ],
        compiler_params=pltpu.CompilerParams(
            dimension_semantics=("arbitrary",)),
    )(theta, q_t.reshape(_D, 1), u.reshape(_N, 1), A_Transfer)


def _sc_topk_body(z_hbm, zb_hbm, out_hbm, z_v, zb_v, o_v):
    wid = jax.lax.axis_index("c") * 16 + jax.lax.axis_index("s")

    @pl.when(wid == 0)
    def _():
        pltpu.sync_copy(z_hbm, z_v)
        pltpu.sync_copy(zb_hbm, zb_v)

        def count_gt(bits):
            tvec = jnp.full((16,), bits, jnp.int32)

            def jstep(j, cvec):
                zj = zb_v[pl.ds(j * 16, 16)]
                return cvec + jnp.where(zj > tvec, jnp.int32(1), jnp.int32(0))

            cvec = jax.lax.fori_loop(
                0, _NV, jstep, jnp.zeros((16,), jnp.int32))
            return jnp.sum(cvec)

        def bstep(_, lohi):
            lo, hi = lohi
            mid = lo + jax.lax.shift_right_logical(hi - lo, 1)
            c = count_gt(mid)
            lt = c < _TOPK
            return (jnp.where(lt, lo, mid + 1), jnp.where(lt, mid, hi))

        lo, hi = jax.lax.fori_loop(
            0, 31, bstep, (jnp.int32(0), jnp.int32(0x7F800000)))
        tbits = lo
        n_gt = count_gt(tbits)
        need = _TOPK - n_gt

        tvec = jnp.full((16,), tbits, jnp.int32)

        def mstep(j, carry):
            svec, eqc = carry
            zj = z_v[pl.ds(j * 16, 16)]
            zbj = zb_v[pl.ds(j * 16, 16)]
            gt = zbj > tvec
            eqi = jnp.where(zbj == tvec, jnp.int32(1), jnp.int32(0))
            rank = (plsc.cumsum(eqi) - eqi) + eqc
            keep = jnp.logical_or(gt, jnp.logical_and(eqi > 0, rank < need))
            mz = jnp.where(keep, zj, 0.0)
            o_v[pl.ds(j * 16, 16)] = mz
            return (svec + mz, eqc + jnp.sum(eqi))

        svec, _ = jax.lax.fori_loop(
            0, _NV, mstep, (jnp.zeros((16,), jnp.float32), jnp.int32(0)))
        total = jnp.sum(svec)
        inv_vec = 1.0 / (jnp.full((16,), total) + 1e-8)

        def nstep(j, c):
            o_v[pl.ds(j * 16, 16)] = o_v[pl.ds(j * 16, 16)] * inv_vec
            return c

        jax.lax.fori_loop(0, _NV, nstep, 0)
        pltpu.sync_copy(o_v, out_hbm)


def _sc_stage(z_pad, z_bits):
    mesh = plsc.VectorSubcoreMesh(core_axis_name="c", subcore_axis_name="s")
    call = pl.kernel(
        _sc_topk_body,
        out_type=jax.ShapeDtypeStruct((_NPAD,), jnp.float32),
        mesh=mesh,
        scratch_types=[
            pltpu.VMEM((_NPAD,), jnp.float32),
            pltpu.VMEM((_NPAD,), jnp.int32),
            pltpu.VMEM((_NPAD,), jnp.float32),
        ],
        compiler_params=pltpu.CompilerParams(needs_layout_passes=False),
    )
    return call(z_pad, z_bits)


def kernel(theta, u, A_Transfer, q_t):
    z_pad, z_bits = _tc_stage(theta, u, A_Transfer, q_t)
    out = _sc_stage(z_pad.reshape(_NPAD), z_bits.reshape(_NPAD))
    return out[:_N]
